# baseline (speedup 1.0000x reference)
import functools

import jax
import jax.numpy as jnp
from jax import lax
from jax.experimental import pallas as pl
from jax.experimental.pallas import tpu as pltpu
from jax.experimental.pallas import tpu_sc as plsc

N = 10000
E = 160000
H = 256
IN = 2 * H
HH = H // 2

NC = 2
NS = 16
K = 128
NB = 79
EPT = NB * K
E_PAD = EPT * NS

NH = N // 2
ACC_R = 5120
ZPT = ACC_R // NS
WPT = 312
WLAST = NH - (NS - 1) * WPT

ROW_BLK = 1000
GRID_I = N // ROW_BLK


def _stage1_body(ih_ref, h_ref, w1_ref, out_ref):
    w = w1_ref[...]
    out_ref[...] = (
        jnp.dot(ih_ref[...], w[:H, :], preferred_element_type=jnp.float32)
        + jnp.dot(h_ref[...], w[H:, :], preferred_element_type=jnp.float32)
    )


def _stage1(input_hidden, hidden, W1):
    return pl.pallas_call(
        _stage1_body,
        grid=(GRID_I, NC),
        in_specs=[
            pl.BlockSpec((ROW_BLK, H), lambda i, j: (i, 0)),
            pl.BlockSpec((ROW_BLK, H), lambda i, j: (i, 0)),
            pl.BlockSpec((IN, HH), lambda i, j: (0, j)),
        ],
        out_specs=pl.BlockSpec((ROW_BLK, HH), lambda i, j: (j * GRID_I + i, 0)),
        out_shape=jax.ShapeDtypeStruct((NC * N, HH), jnp.float32),
    )(input_hidden, hidden, W1)


def _sc_body(y_hbm, src_hbm, dsta_hbm, dstb_hbm, out_hbm,
             srcoff, dsta, dstb, rows, rows1, zbuf, sem, sem1, acc):
    c = lax.axis_index("c")
    s = lax.axis_index("s")

    pltpu.sync_copy(src_hbm.at[s], srcoff)
    pltpu.sync_copy(dsta_hbm.at[s], dsta)
    pltpu.sync_copy(dstb_hbm.at[s], dstb)

    yoff = c * N

    def _off_body(j, carry):
        for i in range(K // 16):
            sl = pl.ds(i * 16, 16)
            srcoff[j, sl] = srcoff[j, sl] + yoff
        return carry

    lax.fori_loop(0, NB, _off_body, 0)

    def _z_body(j, carry):
        for i in range(HH // 16):
            zbuf[j, pl.ds(i * 16, 16)] = jnp.zeros((16,), jnp.float32)
        return carry

    lax.fori_loop(0, K, _z_body, 0)

    ooff = c * N

    def _gather(j, buf, s_):
        pltpu.async_copy(y_hbm.at[srcoff.at[j]], buf, s_)

    def _wait(j, buf, s_):
        pltpu.make_async_copy(y_hbm.at[srcoff.at[j]], buf, s_).wait()

    for p in range(2):
        dref = dsta if p == 0 else dstb

        pltpu.sync_copy(zbuf, acc.at[pl.ds(s * ZPT, K)])
        pltpu.sync_copy(zbuf, acc.at[pl.ds(s * ZPT + K, K)])
        pltpu.sync_copy(zbuf.at[pl.ds(0, ZPT - 2 * K)],
                        acc.at[pl.ds(s * ZPT + 2 * K, ZPT - 2 * K)])
        plsc.subcore_barrier()

        def _scatter(j, buf):
            pltpu.sync_copy(buf, acc.at[dref.at[j]], add=True)

        _gather(0, rows, sem)

        def _main_body(t, carry):
            _gather(2 * t + 1, rows1, sem1)
            _wait(2 * t, rows, sem)
            _scatter(2 * t, rows)
            _gather(2 * t + 2, rows, sem)
            _wait(2 * t + 1, rows1, sem1)
            _scatter(2 * t + 1, rows1)
            return carry

        lax.fori_loop(0, (NB - 1) // 2, _main_body, 0)
        _wait(NB - 1, rows, sem)
        _scatter(NB - 1, rows)

        plsc.subcore_barrier()

        soff = p * 8
        doff = ooff + p * NH

        @pl.when(s < NS - 1)
        def _():
            pltpu.sync_copy(acc.at[pl.ds(soff + s * WPT, WPT)],
                            out_hbm.at[pl.ds(doff + s * WPT, WPT)])

        @pl.when(s == NS - 1)
        def _():
            pltpu.sync_copy(acc.at[pl.ds(soff + (NS - 1) * WPT, WLAST)],
                            out_hbm.at[pl.ds(doff + (NS - 1) * WPT, WLAST)])

        plsc.subcore_barrier()


def _stage2(y_split, src_r, dsta_r, dstb_r):
    mesh = plsc.VectorSubcoreMesh(core_axis_name="c", subcore_axis_name="s")
    f = functools.partial(
        pl.kernel,
        mesh=mesh,
        out_type=jax.ShapeDtypeStruct((NC * N, HH), jnp.float32),
        scratch_types=[
            pltpu.VMEM((NB, K), jnp.int32),
            pltpu.VMEM((NB, K), jnp.int32),
            pltpu.VMEM((NB, K), jnp.int32),
            pltpu.VMEM((K, HH), jnp.float32),
            pltpu.VMEM((K, HH), jnp.float32),
            pltpu.VMEM((K, HH), jnp.float32),
            pltpu.SemaphoreType.DMA,
            pltpu.SemaphoreType.DMA,
            pltpu.VMEM_SHARED((ACC_R, HH), jnp.float32),
        ],
    )(_sc_body)
    return f(y_split, src_r, dsta_r, dstb_r)


def _stage3_body(y0_ref, y1_ref, a0_ref, a1_ref, b1_ref, w2_ref, b2_ref,
                 out_ref):
    z = jnp.concatenate(
        [y0_ref[...] + a0_ref[...], y1_ref[...] + a1_ref[...]], axis=1)
    hmat = jnp.maximum(z + b1_ref[...], 0.0)
    out_ref[...] = (
        jnp.dot(hmat, w2_ref[...], preferred_element_type=jnp.float32)
        + b2_ref[...]
    )


def _stage3(y_split, agg, b1, W2, b2):
    return pl.pallas_call(
        _stage3_body,
        grid=(GRID_I,),
        in_specs=[
            pl.BlockSpec((ROW_BLK, HH), lambda i: (i, 0)),
            pl.BlockSpec((ROW_BLK, HH), lambda i: (GRID_I + i, 0)),
            pl.BlockSpec((ROW_BLK, HH), lambda i: (i, 0)),
            pl.BlockSpec((ROW_BLK, HH), lambda i: (GRID_I + i, 0)),
            pl.BlockSpec((1, H), lambda i: (0, 0)),
            pl.BlockSpec((H, H), lambda i: (0, 0)),
            pl.BlockSpec((1, H), lambda i: (0, 0)),
        ],
        out_specs=pl.BlockSpec((ROW_BLK, H), lambda i: (i, 0)),
        out_shape=jax.ShapeDtypeStruct((N, H), jnp.float32),
    )(y_split, y_split, agg, agg,
      b1.reshape(1, H), W2, b2.reshape(1, H))


def kernel(input_hidden, hidden, last_hidden, batch_assignment, edge_index,
           W1, b1, W2, b2):
    y_split = _stage1(input_hidden, hidden, W1)

    pad = E_PAD - E
    src_p = jnp.concatenate([edge_index[0], jnp.zeros((pad,), jnp.int32)])
    dst_p = jnp.concatenate([edge_index[1], jnp.full((pad,), N, jnp.int32)])
    dsta = jnp.minimum(dst_p, NH)
    dstb = jnp.maximum(dst_p, NH - 8) - (NH - 8)
    src_r = src_p.reshape(NS, NB, K)
    dsta_r = dsta.reshape(NS, NB, K)
    dstb_r = dstb.reshape(NS, NB, K)

    agg = _stage2(y_split, src_r, dsta_r, dstb_r)

    return _stage3(y_split, agg, b1, W2, b2)
